# baseline (speedup 1.0000x reference)
import functools

import jax
import jax.numpy as jnp
from jax import lax
from jax.experimental import pallas as pl
from jax.experimental.pallas import tpu as pltpu
from jax.experimental.pallas import tpu_sc as plsc

L = 16
BLK = 128

_INT_MAX = 2**31 - 1

_SC_PARAMS = dict(
    needs_layout_passes=False,
    use_tc_tiling_on_sc=True,
    disable_bounds_checks=True,
)


def _worker_id():
    return lax.axis_index("s") * lax.axis_size("c") + lax.axis_index("c")


def _take16(v, idx):
    return lax.gather(
        v, idx[:, None],
        dimension_numbers=lax.GatherDimensionNumbers(
            offset_dims=(), collapsed_slice_dims=(0,), start_index_map=(0,)),
        slice_sizes=(1,),
        mode=lax.GatherScatterMode.PROMISE_IN_BOUNDS)


def _better(val, idx, best_val, best_idx):
    upd = (val < best_val) | ((val == best_val) & (idx < best_idx))
    return jnp.where(upd, val, best_val), jnp.where(upd, idx, best_idx)


def _make_stage1(K, NW, CH_B, TB):
    NB = -(-K // BLK)
    NBW = NB - TB
    BASE_B = NBW // NW
    EXTRA = NBW % NW
    NCHUNK = -(-(BASE_B + (1 if EXTRA else 0)) // CH_B)
    CH_C = CH_B * BLK

    mesh = plsc.VectorSubcoreMesh(core_axis_name="c", subcore_axis_name="s")

    @functools.partial(
        pl.kernel,
        out_type=(
            jax.ShapeDtypeStruct((NW, L), jnp.float32),
            jax.ShapeDtypeStruct((NW, L), jnp.int32),
        ),
        mesh=mesh,
        compiler_params=pltpu.CompilerParams(**_SC_PARAMS),
        scratch_types=[
            pltpu.VMEM((L, CH_C), jnp.float32),
            pltpu.VMEM((L, CH_C), jnp.float32),
            pltpu.VMEM((1, L), jnp.float32),
            pltpu.VMEM((L,), jnp.float32),
            pltpu.VMEM((L,), jnp.int32),
            pltpu.SemaphoreType.DMA,
            pltpu.SemaphoreType.DMA,
        ],
    )
    def stage1(q_hbm, obs_hbm, oval_hbm, oidx_hbm,
               buf0, buf1, qv, sval, sidx, sem0, sem1):
        w = _worker_id()
        b0 = TB + w * BASE_B + jnp.minimum(w, EXTRA)
        nb = BASE_B + jnp.where(w < EXTRA, 1, 0)
        col0 = b0 * BLK
        col_hi = (b0 + nb) * BLK - CH_C

        pltpu.sync_copy(q_hbm, qv)
        qvec = qv[0]
        qs = [jnp.full((L,), qvec[f], jnp.float32) for f in range(L)]

        iota = lax.iota(jnp.int32, L)

        bufs = (buf0, buf1)
        sems = (sem0, sem1)

        def chunk_base(c):
            return jnp.minimum(col0 + c * CH_C, col_hi)

        def start(c):
            return pltpu.async_copy(
                obs_hbm.at[:, pl.ds(chunk_base(c), CH_C)], bufs[c % 2],
                sems[c % 2])

        best_val = jnp.full((L,), jnp.inf, jnp.float32)
        best_idx = jnp.zeros((L,), jnp.int32)

        inf16 = jnp.full((L,), jnp.inf, jnp.float32)

        def process(c, buf, sem, bv, bi):
            pltpu.make_async_copy(
                obs_hbm.at[:, pl.ds(chunk_base(c), CH_C)], buf, sem).wait()
            cb = chunk_base(c)

            @pl.when(cb + CH_C > K)
            def _():
                def poison(t, _):
                    for f in range(L):
                        buf[f, pl.ds(K - cb + t * L, L)] = inf16
                    return 0
                lax.fori_loop(0, (cb + CH_C - K) // L, poison, 0)

            base_idx = cb + iota

            def group(g, carry):
                gv, gi = carry
                r = g * L
                parts = []
                for f in range(L):
                    t = buf[f, pl.ds(r, L)] - qs[f]
                    parts.append(t * t)
                while len(parts) > 1:
                    parts = [parts[i] + parts[i + 1]
                             for i in range(0, len(parts), 2)]
                dist = parts[0]
                upd = dist < gv
                return (jnp.minimum(dist, gv),
                        jnp.where(upd, base_idx + r, gi))

            return lax.fori_loop(0, CH_C // L, group, (bv, bi))

        cp0 = start(0)
        cp1 = start(1)

        def pair(i, carry):
            bv, bi = carry
            c0 = 2 * i
            bv, bi = process(c0, buf0, sem0, bv, bi)

            @pl.when(c0 + 2 < NCHUNK)
            def _():
                pltpu.async_copy(
                    obs_hbm.at[:, pl.ds(chunk_base(c0 + 2), CH_C)], buf0,
                    sem0)

            bv, bi = process(c0 + 1, buf1, sem1, bv, bi)

            @pl.when(c0 + 3 < NCHUNK)
            def _():
                pltpu.async_copy(
                    obs_hbm.at[:, pl.ds(chunk_base(c0 + 3), CH_C)], buf1,
                    sem1)

            return bv, bi

        best_val, best_idx = lax.fori_loop(
            0, NCHUNK // 2, pair, (best_val, best_idx))
        if NCHUNK % 2:
            best_val, best_idx = process(
                NCHUNK - 1, buf0, sem0, best_val, best_idx)

        sval[...] = best_val
        sidx[...] = best_idx
        pltpu.sync_copy(sval, oval_hbm.at[w])
        pltpu.sync_copy(sidx, oidx_hbm.at[w])

    return stage1


def _make_tc_scan(K, TB, BN):
    TCOLS = TB * BLK
    assert TCOLS % BN == 0 and TCOLS <= K
    NSTEP = TCOLS // BN
    SUB = 8

    def scan(q_ref, blk_ref, oval_ref, oidx_ref, bestv, besti):
        i = pl.program_id(0)

        @pl.when(i == 0)
        def _():
            bestv[...] = jnp.full((SUB, BLK), jnp.inf, jnp.float32)
            besti[...] = jnp.zeros((SUB, BLK), jnp.int32)

        t = blk_ref[...] - q_ref[...]
        d = jnp.sum(t * t, axis=0).reshape(BN // BLK, BLK)
        col8 = (lax.broadcasted_iota(jnp.int32, (SUB, BLK), 0) * BLK
                + lax.broadcasted_iota(jnp.int32, (SUB, BLK), 1))
        idx8 = i * BN + col8
        bv = bestv[...]
        upd = d < bv
        bestv[...] = jnp.minimum(d, bv)
        besti[...] = jnp.where(upd, idx8, besti[...])

        @pl.when(i == NSTEP - 1)
        def _():
            oval_ref[...] = bestv[...]
            oidx_ref[...] = besti[...]

    return pl.pallas_call(
        scan,
        grid=(NSTEP,),
        out_shape=(
            jax.ShapeDtypeStruct((SUB, BLK), jnp.float32),
            jax.ShapeDtypeStruct((SUB, BLK), jnp.int32),
        ),
        in_specs=[
            pl.BlockSpec((L, 1), lambda i: (0, 0)),
            pl.BlockSpec((L, BN), lambda i: (0, i)),
        ],
        out_specs=(
            pl.BlockSpec((SUB, BLK), lambda i: (0, 0)),
            pl.BlockSpec((SUB, BLK), lambda i: (0, 0)),
        ),
        scratch_shapes=[
            pltpu.VMEM((SUB, BLK), jnp.float32),
            pltpu.VMEM((SUB, BLK), jnp.int32),
        ],
        compiler_params=pltpu.CompilerParams(
            dimension_semantics=("arbitrary",)),
    )


def _make_stage2(K, NW):
    def stage2(vals_ref, idxs_ref, tval_ref, tidx_ref, pv_hbm, pm_hbm,
               ovel_ref, omask_ref, blkv, blkm, sem):
        v = vals_ref[...]
        ix = idxs_ref[...]
        m = jnp.min(v)
        ind = jnp.min(jnp.where(v == m, ix, _INT_MAX))
        tv = tval_ref[...]
        tm_ = jnp.min(tv)
        tind = jnp.min(jnp.where(tv == tm_, tidx_ref[...], _INT_MAX))
        tc_wins = (tm_ < m) | ((tm_ == m) & (tind < ind))
        ind = jnp.where(tc_wins, tind, ind)
        blk0 = (ind // BLK) * BLK
        off = ind - blk0
        pltpu.make_async_copy(
            pv_hbm.at[:, pl.ds(blk0, BLK)], blkv, sem).start()
        pltpu.make_async_copy(
            pv_hbm.at[:, pl.ds(blk0, BLK)], blkv, sem).wait()
        pltpu.make_async_copy(
            pm_hbm.at[:, pl.ds(blk0, BLK)], blkm, sem).start()
        pltpu.make_async_copy(
            pm_hbm.at[:, pl.ds(blk0, BLK)], blkm, sem).wait()
        lane = lax.broadcasted_iota(jnp.int32, (L, BLK), 1)
        sel = lane == off
        rowv = jnp.sum(jnp.where(sel, blkv[...], 0.0), axis=1)
        rowm = jnp.sum(jnp.where(sel, blkm[...], 0.0), axis=1)
        ovel_ref[...] = rowv.reshape(1, L)
        omask_ref[...] = rowm.reshape(1, L)

    return pl.pallas_call(
        stage2,
        out_shape=(
            jax.ShapeDtypeStruct((1, L), jnp.float32),
            jax.ShapeDtypeStruct((1, L), jnp.float32),
        ),
        in_specs=[
            pl.BlockSpec(memory_space=pltpu.MemorySpace.VMEM),
            pl.BlockSpec(memory_space=pltpu.MemorySpace.VMEM),
            pl.BlockSpec(memory_space=pltpu.MemorySpace.VMEM),
            pl.BlockSpec(memory_space=pltpu.MemorySpace.VMEM),
            pl.BlockSpec(memory_space=pl.ANY),
            pl.BlockSpec(memory_space=pl.ANY),
        ],
        out_specs=(
            pl.BlockSpec(memory_space=pltpu.MemorySpace.VMEM),
            pl.BlockSpec(memory_space=pltpu.MemorySpace.VMEM),
        ),
        scratch_shapes=[
            pltpu.VMEM((L, BLK), jnp.float32),
            pltpu.VMEM((L, BLK), jnp.float32),
            pltpu.SemaphoreType.DMA,
        ],
        compiler_params=pltpu.CompilerParams(disable_bounds_checks=True),
    )


def kernel(in_vel, obs_vel, pred_vel, pred_mask):
    K, D = obs_vel.shape
    assert D == L
    info = plsc.get_sparse_core_info()
    NW = info.num_cores * info.num_subcores
    CH_B = 16
    TB = 3280

    obs_t = obs_vel.T
    pv_t = pred_vel.T
    pm_t = pred_mask.T

    vals, idxs = _make_stage1(K, NW, CH_B, TB)(in_vel, obs_t)
    tval, tidx = _make_tc_scan(K, TB, 1024)(in_vel.T, obs_t)
    best_vel, best_mask = _make_stage2(K, NW)(
        vals, idxs, tval, tidx, pv_t, pm_t)
    return best_vel, best_mask


# speedup vs baseline: 2.6683x; 2.6683x over previous
import functools

import jax
import jax.numpy as jnp
from jax import lax
from jax.experimental import pallas as pl
from jax.experimental.pallas import tpu as pltpu
from jax.experimental.pallas import tpu_sc as plsc

L = 16
BLK = 128

_INT_MAX = 2**31 - 1

_SC_PARAMS = dict(
    needs_layout_passes=False,
    use_tc_tiling_on_sc=True,
    disable_bounds_checks=True,
)


def _worker_id():
    return lax.axis_index("s") * lax.axis_size("c") + lax.axis_index("c")


def _take16(v, idx):
    return lax.gather(
        v, idx[:, None],
        dimension_numbers=lax.GatherDimensionNumbers(
            offset_dims=(), collapsed_slice_dims=(0,), start_index_map=(0,)),
        slice_sizes=(1,),
        mode=lax.GatherScatterMode.PROMISE_IN_BOUNDS)


def _better(val, idx, best_val, best_idx):
    upd = (val < best_val) | ((val == best_val) & (idx < best_idx))
    return jnp.where(upd, val, best_val), jnp.where(upd, idx, best_idx)


def _make_stage1(K, NW, CH_B, TB):
    NB = -(-K // BLK)
    NBW = NB - TB
    BASE_B = NBW // NW
    EXTRA = NBW % NW
    NCHUNK = -(-(BASE_B + (1 if EXTRA else 0)) // CH_B)
    CH_C = CH_B * BLK

    mesh = plsc.VectorSubcoreMesh(core_axis_name="c", subcore_axis_name="s")

    @functools.partial(
        pl.kernel,
        out_type=(
            jax.ShapeDtypeStruct((NW, L), jnp.float32),
            jax.ShapeDtypeStruct((NW, L), jnp.int32),
        ),
        mesh=mesh,
        compiler_params=pltpu.CompilerParams(**_SC_PARAMS),
        scratch_types=[
            pltpu.VMEM((L, CH_C), jnp.float32),
            pltpu.VMEM((L, CH_C), jnp.float32),
            pltpu.VMEM((1, L), jnp.float32),
            pltpu.VMEM((L,), jnp.float32),
            pltpu.VMEM((L,), jnp.int32),
            pltpu.SemaphoreType.DMA,
            pltpu.SemaphoreType.DMA,
        ],
    )
    def stage1(q_hbm, obs_hbm, oval_hbm, oidx_hbm,
               buf0, buf1, qv, sval, sidx, sem0, sem1):
        w = _worker_id()
        b0 = TB + w * BASE_B + jnp.minimum(w, EXTRA)
        nb = BASE_B + jnp.where(w < EXTRA, 1, 0)
        col0 = b0 * BLK
        col_hi = (b0 + nb) * BLK - CH_C

        pltpu.sync_copy(q_hbm, qv)
        qvec = qv[0]
        qs = [jnp.full((L,), qvec[f], jnp.float32) for f in range(L)]

        iota = lax.iota(jnp.int32, L)

        bufs = (buf0, buf1)
        sems = (sem0, sem1)

        def chunk_base(c):
            return jnp.minimum(col0 + c * CH_C, col_hi)

        def start(c):
            return pltpu.async_copy(
                obs_hbm.at[:, pl.ds(chunk_base(c), CH_C)], bufs[c % 2],
                sems[c % 2])

        best_val = jnp.full((L,), jnp.inf, jnp.float32)
        best_idx = jnp.zeros((L,), jnp.int32)

        inf16 = jnp.full((L,), jnp.inf, jnp.float32)

        def process(c, buf, sem, bv, bi):
            pltpu.make_async_copy(
                obs_hbm.at[:, pl.ds(chunk_base(c), CH_C)], buf, sem).wait()
            cb = chunk_base(c)

            @pl.when(cb + CH_C > K)
            def _():
                def poison(t, _):
                    for f in range(L):
                        buf[f, pl.ds(K - cb + t * L, L)] = inf16
                    return 0
                lax.fori_loop(0, (cb + CH_C - K) // L, poison, 0)

            base_idx = cb + iota

            def group(g, carry):
                gv, gi = carry
                r = g * L
                parts = []
                for f in range(L):
                    t = buf[f, pl.ds(r, L)] - qs[f]
                    parts.append(t * t)
                while len(parts) > 1:
                    parts = [parts[i] + parts[i + 1]
                             for i in range(0, len(parts), 2)]
                dist = parts[0]
                upd = dist < gv
                return (jnp.minimum(dist, gv),
                        jnp.where(upd, base_idx + r, gi))

            return lax.fori_loop(0, CH_C // L, group, (bv, bi))

        cp0 = start(0)
        cp1 = start(1)

        def pair(i, carry):
            bv, bi = carry
            c0 = 2 * i
            bv, bi = process(c0, buf0, sem0, bv, bi)

            @pl.when(c0 + 2 < NCHUNK)
            def _():
                pltpu.async_copy(
                    obs_hbm.at[:, pl.ds(chunk_base(c0 + 2), CH_C)], buf0,
                    sem0)

            bv, bi = process(c0 + 1, buf1, sem1, bv, bi)

            @pl.when(c0 + 3 < NCHUNK)
            def _():
                pltpu.async_copy(
                    obs_hbm.at[:, pl.ds(chunk_base(c0 + 3), CH_C)], buf1,
                    sem1)

            return bv, bi

        best_val, best_idx = lax.fori_loop(
            0, NCHUNK // 2, pair, (best_val, best_idx))
        if NCHUNK % 2:
            best_val, best_idx = process(
                NCHUNK - 1, buf0, sem0, best_val, best_idx)

        sval[...] = best_val
        sidx[...] = best_idx
        pltpu.sync_copy(sval, oval_hbm.at[w])
        pltpu.sync_copy(sidx, oidx_hbm.at[w])

    return stage1


def _make_tc_scan(K, TB, BN):
    TCOLS = TB * BLK
    assert TCOLS % BN == 0 and TCOLS <= K
    NSTEP = TCOLS // BN

    def scan(q_ref, blk_ref, oval_ref, oidx_ref, bestv, besti):
        i = pl.program_id(0)

        @pl.when(i == 0)
        def _():
            bestv[...] = jnp.full((1, BN), jnp.inf, jnp.float32)
            besti[...] = jnp.zeros((1, BN), jnp.int32)

        t = blk_ref[...] - q_ref[...]
        d = jnp.sum(t * t, axis=0, keepdims=True)
        idx = i * BN + lax.broadcasted_iota(jnp.int32, (1, BN), 1)
        bv = bestv[...]
        upd = d < bv
        bestv[...] = jnp.minimum(d, bv)
        besti[...] = jnp.where(upd, idx, besti[...])

        @pl.when(i == NSTEP - 1)
        def _():
            oval_ref[...] = bestv[...]
            oidx_ref[...] = besti[...]

    return pl.pallas_call(
        scan,
        grid=(NSTEP,),
        out_shape=(
            jax.ShapeDtypeStruct((1, BN), jnp.float32),
            jax.ShapeDtypeStruct((1, BN), jnp.int32),
        ),
        in_specs=[
            pl.BlockSpec((L, 1), lambda i: (0, 0)),
            pl.BlockSpec((L, BN), lambda i: (0, i)),
        ],
        out_specs=(
            pl.BlockSpec((1, BN), lambda i: (0, 0)),
            pl.BlockSpec((1, BN), lambda i: (0, 0)),
        ),
        scratch_shapes=[
            pltpu.VMEM((1, BN), jnp.float32),
            pltpu.VMEM((1, BN), jnp.int32),
        ],
        compiler_params=pltpu.CompilerParams(
            dimension_semantics=("arbitrary",)),
    )


def _make_stage2(K, NW):
    def stage2(vals_ref, idxs_ref, tval_ref, tidx_ref, pv_hbm, pm_hbm,
               ovel_ref, omask_ref, blkv, blkm, sem):
        v = vals_ref[...]
        ix = idxs_ref[...]
        m = jnp.min(v)
        ind = jnp.min(jnp.where(v == m, ix, _INT_MAX))
        tv = tval_ref[...]
        tm_ = jnp.min(tv)
        tind = jnp.min(jnp.where(tv == tm_, tidx_ref[...], _INT_MAX))
        tc_wins = (tm_ < m) | ((tm_ == m) & (tind < ind))
        ind = jnp.where(tc_wins, tind, ind)
        blk0 = (ind // BLK) * BLK
        off = ind - blk0
        pltpu.make_async_copy(
            pv_hbm.at[:, pl.ds(blk0, BLK)], blkv, sem).start()
        pltpu.make_async_copy(
            pv_hbm.at[:, pl.ds(blk0, BLK)], blkv, sem).wait()
        pltpu.make_async_copy(
            pm_hbm.at[:, pl.ds(blk0, BLK)], blkm, sem).start()
        pltpu.make_async_copy(
            pm_hbm.at[:, pl.ds(blk0, BLK)], blkm, sem).wait()
        lane = lax.broadcasted_iota(jnp.int32, (L, BLK), 1)
        sel = lane == off
        rowv = jnp.sum(jnp.where(sel, blkv[...], 0.0), axis=1)
        rowm = jnp.sum(jnp.where(sel, blkm[...], 0.0), axis=1)
        ovel_ref[...] = rowv.reshape(1, L)
        omask_ref[...] = rowm.reshape(1, L)

    return pl.pallas_call(
        stage2,
        out_shape=(
            jax.ShapeDtypeStruct((1, L), jnp.float32),
            jax.ShapeDtypeStruct((1, L), jnp.float32),
        ),
        in_specs=[
            pl.BlockSpec(memory_space=pltpu.MemorySpace.VMEM),
            pl.BlockSpec(memory_space=pltpu.MemorySpace.VMEM),
            pl.BlockSpec(memory_space=pltpu.MemorySpace.VMEM),
            pl.BlockSpec(memory_space=pltpu.MemorySpace.VMEM),
            pl.BlockSpec(memory_space=pl.ANY),
            pl.BlockSpec(memory_space=pl.ANY),
        ],
        out_specs=(
            pl.BlockSpec(memory_space=pltpu.MemorySpace.VMEM),
            pl.BlockSpec(memory_space=pltpu.MemorySpace.VMEM),
        ),
        scratch_shapes=[
            pltpu.VMEM((L, BLK), jnp.float32),
            pltpu.VMEM((L, BLK), jnp.float32),
            pltpu.SemaphoreType.DMA,
        ],
        compiler_params=pltpu.CompilerParams(disable_bounds_checks=True),
    )


def kernel(in_vel, obs_vel, pred_vel, pred_mask):
    K, D = obs_vel.shape
    assert D == L
    info = plsc.get_sparse_core_info()
    NW = info.num_cores * info.num_subcores
    CH_B = 16
    TB = 3264

    obs_t = obs_vel.T
    pv_t = pred_vel.T
    pm_t = pred_mask.T

    vals, idxs = _make_stage1(K, NW, CH_B, TB)(in_vel, obs_t)
    tval, tidx = _make_tc_scan(K, TB, 4096)(in_vel.T, obs_t)
    best_vel, best_mask = _make_stage2(K, NW)(
        vals, idxs, tval, tidx, pv_t, pm_t)
    return best_vel, best_mask


# speedup vs baseline: 4.1231x; 1.5452x over previous
import functools

import jax
import jax.numpy as jnp
from jax import lax
from jax.experimental import pallas as pl
from jax.experimental.pallas import tpu as pltpu
from jax.experimental.pallas import tpu_sc as plsc

L = 16
BLK = 128

_INT_MAX = 2**31 - 1

_SC_PARAMS = dict(
    needs_layout_passes=False,
    use_tc_tiling_on_sc=True,
    disable_bounds_checks=True,
)


def _worker_id():
    return lax.axis_index("s") * lax.axis_size("c") + lax.axis_index("c")


def _take16(v, idx):
    return lax.gather(
        v, idx[:, None],
        dimension_numbers=lax.GatherDimensionNumbers(
            offset_dims=(), collapsed_slice_dims=(0,), start_index_map=(0,)),
        slice_sizes=(1,),
        mode=lax.GatherScatterMode.PROMISE_IN_BOUNDS)


def _better(val, idx, best_val, best_idx):
    upd = (val < best_val) | ((val == best_val) & (idx < best_idx))
    return jnp.where(upd, val, best_val), jnp.where(upd, idx, best_idx)


def _make_stage1(K, NW, CH_B, TB):
    NB = -(-K // BLK)
    NBW = NB - TB
    BASE_B = NBW // NW
    EXTRA = NBW % NW
    NCHUNK = -(-(BASE_B + (1 if EXTRA else 0)) // CH_B)
    CH_C = CH_B * BLK

    mesh = plsc.VectorSubcoreMesh(core_axis_name="c", subcore_axis_name="s")

    @functools.partial(
        pl.kernel,
        out_type=(
            jax.ShapeDtypeStruct((NW, L), jnp.float32),
            jax.ShapeDtypeStruct((NW, L), jnp.int32),
        ),
        mesh=mesh,
        compiler_params=pltpu.CompilerParams(**_SC_PARAMS),
        scratch_types=[
            pltpu.VMEM((L, CH_C), jnp.float32),
            pltpu.VMEM((L, CH_C), jnp.float32),
            pltpu.VMEM((1, L), jnp.float32),
            pltpu.VMEM((L,), jnp.float32),
            pltpu.VMEM((L,), jnp.int32),
            pltpu.SemaphoreType.DMA,
            pltpu.SemaphoreType.DMA,
        ],
    )
    def stage1(q_hbm, obs_hbm, oval_hbm, oidx_hbm,
               buf0, buf1, qv, sval, sidx, sem0, sem1):
        w = _worker_id()
        b0 = TB + w * BASE_B + jnp.minimum(w, EXTRA)
        nb = BASE_B + jnp.where(w < EXTRA, 1, 0)
        col0 = b0 * BLK
        col_hi = (b0 + nb) * BLK - CH_C

        pltpu.sync_copy(q_hbm, qv)
        qvec = qv[0]
        qs = [jnp.full((L,), qvec[f], jnp.float32) for f in range(L)]

        iota = lax.iota(jnp.int32, L)

        bufs = (buf0, buf1)
        sems = (sem0, sem1)

        def chunk_base(c):
            return jnp.minimum(col0 + c * CH_C, col_hi)

        def start(c):
            return pltpu.async_copy(
                obs_hbm.at[:, pl.ds(chunk_base(c), CH_C)], bufs[c % 2],
                sems[c % 2])

        best_val = jnp.full((L,), jnp.inf, jnp.float32)
        best_idx = jnp.zeros((L,), jnp.int32)

        inf16 = jnp.full((L,), jnp.inf, jnp.float32)

        def process(c, buf, sem, bv, bi):
            pltpu.make_async_copy(
                obs_hbm.at[:, pl.ds(chunk_base(c), CH_C)], buf, sem).wait()
            cb = chunk_base(c)

            @pl.when(cb + CH_C > K)
            def _():
                def poison(t, _):
                    for f in range(L):
                        buf[f, pl.ds(K - cb + t * L, L)] = inf16
                    return 0
                lax.fori_loop(0, (cb + CH_C - K) // L, poison, 0)

            base_idx = cb + iota

            def group(g, carry):
                gv, gi = carry
                r = g * L
                parts = []
                for f in range(L):
                    t = buf[f, pl.ds(r, L)] - qs[f]
                    parts.append(t * t)
                while len(parts) > 1:
                    parts = [parts[i] + parts[i + 1]
                             for i in range(0, len(parts), 2)]
                dist = parts[0]
                upd = dist < gv
                return (jnp.minimum(dist, gv),
                        jnp.where(upd, base_idx + r, gi))

            return lax.fori_loop(0, CH_C // L, group, (bv, bi))

        cp0 = start(0)
        cp1 = start(1)

        def pair(i, carry):
            bv, bi = carry
            c0 = 2 * i
            bv, bi = process(c0, buf0, sem0, bv, bi)

            @pl.when(c0 + 2 < NCHUNK)
            def _():
                pltpu.async_copy(
                    obs_hbm.at[:, pl.ds(chunk_base(c0 + 2), CH_C)], buf0,
                    sem0)

            bv, bi = process(c0 + 1, buf1, sem1, bv, bi)

            @pl.when(c0 + 3 < NCHUNK)
            def _():
                pltpu.async_copy(
                    obs_hbm.at[:, pl.ds(chunk_base(c0 + 3), CH_C)], buf1,
                    sem1)

            return bv, bi

        best_val, best_idx = lax.fori_loop(
            0, NCHUNK // 2, pair, (best_val, best_idx))
        if NCHUNK % 2:
            best_val, best_idx = process(
                NCHUNK - 1, buf0, sem0, best_val, best_idx)

        sval[...] = best_val
        sidx[...] = best_idx
        pltpu.sync_copy(sval, oval_hbm.at[w])
        pltpu.sync_copy(sidx, oidx_hbm.at[w])

    return stage1


def _make_tc_scan(K, TB, BN):
    TCOLS = TB * BLK
    assert TCOLS % BN == 0 and TCOLS <= K
    NSTEP = TCOLS // BN

    def scan(q_ref, blk_ref, oval_ref, oidx_ref, bestv, besti):
        i = pl.program_id(0)

        @pl.when(i == 0)
        def _():
            bestv[...] = jnp.full((1, BN), jnp.inf, jnp.float32)
            besti[...] = jnp.zeros((1, BN), jnp.int32)

        t = blk_ref[...] - q_ref[...]
        d = jnp.sum(t * t, axis=0, keepdims=True)
        idx = i * BN + lax.broadcasted_iota(jnp.int32, (1, BN), 1)
        bv = bestv[...]
        upd = d < bv
        bestv[...] = jnp.minimum(d, bv)
        besti[...] = jnp.where(upd, idx, besti[...])

        @pl.when(i == NSTEP - 1)
        def _():
            oval_ref[...] = bestv[...]
            oidx_ref[...] = besti[...]

    return pl.pallas_call(
        scan,
        grid=(NSTEP,),
        out_shape=(
            jax.ShapeDtypeStruct((1, BN), jnp.float32),
            jax.ShapeDtypeStruct((1, BN), jnp.int32),
        ),
        in_specs=[
            pl.BlockSpec((L, 1), lambda i: (0, 0)),
            pl.BlockSpec((L, BN), lambda i: (0, i)),
        ],
        out_specs=(
            pl.BlockSpec((1, BN), lambda i: (0, 0)),
            pl.BlockSpec((1, BN), lambda i: (0, 0)),
        ),
        scratch_shapes=[
            pltpu.VMEM((1, BN), jnp.float32),
            pltpu.VMEM((1, BN), jnp.int32),
        ],
        compiler_params=pltpu.CompilerParams(
            dimension_semantics=("arbitrary",)),
    )


def _make_stage2(K, NW):
    def stage2(vals_ref, idxs_ref, tval_ref, tidx_ref, pv_hbm, pm_hbm,
               ovel_ref, omask_ref, blkv, blkm, sem):
        v = vals_ref[...]
        ix = idxs_ref[...]
        m = jnp.min(v)
        ind = jnp.min(jnp.where(v == m, ix, _INT_MAX))
        tv = tval_ref[...]
        tm_ = jnp.min(tv)
        tind = jnp.min(jnp.where(tv == tm_, tidx_ref[...], _INT_MAX))
        tc_wins = (tm_ < m) | ((tm_ == m) & (tind < ind))
        ind = jnp.where(tc_wins, tind, ind)
        blk0 = (ind // BLK) * BLK
        off = ind - blk0
        pltpu.make_async_copy(
            pv_hbm.at[:, pl.ds(blk0, BLK)], blkv, sem).start()
        pltpu.make_async_copy(
            pv_hbm.at[:, pl.ds(blk0, BLK)], blkv, sem).wait()
        pltpu.make_async_copy(
            pm_hbm.at[:, pl.ds(blk0, BLK)], blkm, sem).start()
        pltpu.make_async_copy(
            pm_hbm.at[:, pl.ds(blk0, BLK)], blkm, sem).wait()
        lane = lax.broadcasted_iota(jnp.int32, (L, BLK), 1)
        sel = lane == off
        rowv = jnp.sum(jnp.where(sel, blkv[...], 0.0), axis=1)
        rowm = jnp.sum(jnp.where(sel, blkm[...], 0.0), axis=1)
        ovel_ref[...] = rowv.reshape(1, L)
        omask_ref[...] = rowm.reshape(1, L)

    return pl.pallas_call(
        stage2,
        out_shape=(
            jax.ShapeDtypeStruct((1, L), jnp.float32),
            jax.ShapeDtypeStruct((1, L), jnp.float32),
        ),
        in_specs=[
            pl.BlockSpec(memory_space=pltpu.MemorySpace.VMEM),
            pl.BlockSpec(memory_space=pltpu.MemorySpace.VMEM),
            pl.BlockSpec(memory_space=pltpu.MemorySpace.VMEM),
            pl.BlockSpec(memory_space=pltpu.MemorySpace.VMEM),
            pl.BlockSpec(memory_space=pl.ANY),
            pl.BlockSpec(memory_space=pl.ANY),
        ],
        out_specs=(
            pl.BlockSpec(memory_space=pltpu.MemorySpace.VMEM),
            pl.BlockSpec(memory_space=pltpu.MemorySpace.VMEM),
        ),
        scratch_shapes=[
            pltpu.VMEM((L, BLK), jnp.float32),
            pltpu.VMEM((L, BLK), jnp.float32),
            pltpu.SemaphoreType.DMA,
        ],
        compiler_params=pltpu.CompilerParams(disable_bounds_checks=True),
    )


def kernel(in_vel, obs_vel, pred_vel, pred_mask):
    K, D = obs_vel.shape
    assert D == L
    info = plsc.get_sparse_core_info()
    NW = info.num_cores * info.num_subcores
    CH_B = 16
    TB = 1568

    obs_t = obs_vel.T
    pv_t = pred_vel.T
    pm_t = pred_mask.T

    vals, idxs = _make_stage1(K, NW, CH_B, TB)(in_vel, obs_t)
    tval, tidx = _make_tc_scan(K, TB, 4096)(in_vel.T, obs_t)
    best_vel, best_mask = _make_stage2(K, NW)(
        vals, idxs, tval, tidx, pv_t, pm_t)
    return best_vel, best_mask


# speedup vs baseline: 4.3051x; 1.0441x over previous
import functools

import jax
import jax.numpy as jnp
from jax import lax
from jax.experimental import pallas as pl
from jax.experimental.pallas import tpu as pltpu
from jax.experimental.pallas import tpu_sc as plsc

L = 16
BLK = 128

_INT_MAX = 2**31 - 1

_SC_PARAMS = dict(
    needs_layout_passes=False,
    use_tc_tiling_on_sc=True,
    disable_bounds_checks=True,
)


def _worker_id():
    return lax.axis_index("s") * lax.axis_size("c") + lax.axis_index("c")


def _take16(v, idx):
    return lax.gather(
        v, idx[:, None],
        dimension_numbers=lax.GatherDimensionNumbers(
            offset_dims=(), collapsed_slice_dims=(0,), start_index_map=(0,)),
        slice_sizes=(1,),
        mode=lax.GatherScatterMode.PROMISE_IN_BOUNDS)


def _better(val, idx, best_val, best_idx):
    upd = (val < best_val) | ((val == best_val) & (idx < best_idx))
    return jnp.where(upd, val, best_val), jnp.where(upd, idx, best_idx)


def _make_stage1(K, NW, CH_B, TB):
    NB = -(-K // BLK)
    NBW = NB - TB
    BASE_B = NBW // NW
    EXTRA = NBW % NW
    NCHUNK = -(-(BASE_B + (1 if EXTRA else 0)) // CH_B)
    CH_C = CH_B * BLK

    mesh = plsc.VectorSubcoreMesh(core_axis_name="c", subcore_axis_name="s")

    @functools.partial(
        pl.kernel,
        out_type=(
            jax.ShapeDtypeStruct((NW, L), jnp.float32),
            jax.ShapeDtypeStruct((NW, L), jnp.int32),
        ),
        mesh=mesh,
        compiler_params=pltpu.CompilerParams(**_SC_PARAMS),
        scratch_types=[
            pltpu.VMEM((L, CH_C), jnp.float32),
            pltpu.VMEM((L, CH_C), jnp.float32),
            pltpu.VMEM((1, L), jnp.float32),
            pltpu.VMEM((L,), jnp.float32),
            pltpu.VMEM((L,), jnp.int32),
            pltpu.SemaphoreType.DMA,
            pltpu.SemaphoreType.DMA,
        ],
    )
    def stage1(q_hbm, obs_hbm, oval_hbm, oidx_hbm,
               buf0, buf1, qv, sval, sidx, sem0, sem1):
        w = _worker_id()
        b0 = TB + w * BASE_B + jnp.minimum(w, EXTRA)
        nb = BASE_B + jnp.where(w < EXTRA, 1, 0)
        col0 = b0 * BLK
        col_hi = (b0 + nb) * BLK - CH_C

        pltpu.sync_copy(q_hbm, qv)
        qvec = qv[0]
        qs = [jnp.full((L,), qvec[f], jnp.float32) for f in range(L)]

        iota = lax.iota(jnp.int32, L)

        bufs = (buf0, buf1)
        sems = (sem0, sem1)

        def chunk_base(c):
            return jnp.minimum(col0 + c * CH_C, col_hi)

        def start(c):
            return pltpu.async_copy(
                obs_hbm.at[:, pl.ds(chunk_base(c), CH_C)], bufs[c % 2],
                sems[c % 2])

        best_val = jnp.full((L,), jnp.inf, jnp.float32)
        best_idx = jnp.zeros((L,), jnp.int32)

        inf16 = jnp.full((L,), jnp.inf, jnp.float32)

        def process(c, buf, sem, bv, bi):
            pltpu.make_async_copy(
                obs_hbm.at[:, pl.ds(chunk_base(c), CH_C)], buf, sem).wait()
            cb = chunk_base(c)

            @pl.when(cb + CH_C > K)
            def _():
                def poison(t, _):
                    for f in range(L):
                        buf[f, pl.ds(K - cb + t * L, L)] = inf16
                    return 0
                lax.fori_loop(0, (cb + CH_C - K) // L, poison, 0)

            base_idx = cb + iota

            def group(g, carry):
                gv, gi = carry
                r = g * L
                parts = []
                for f in range(L):
                    t = buf[f, pl.ds(r, L)] - qs[f]
                    parts.append(t * t)
                while len(parts) > 1:
                    parts = [parts[i] + parts[i + 1]
                             for i in range(0, len(parts), 2)]
                dist = parts[0]
                upd = dist < gv
                return (jnp.minimum(dist, gv),
                        jnp.where(upd, base_idx + r, gi))

            return lax.fori_loop(0, CH_C // L, group, (bv, bi))

        cp0 = start(0)
        cp1 = start(1)

        def pair(i, carry):
            bv, bi = carry
            c0 = 2 * i
            bv, bi = process(c0, buf0, sem0, bv, bi)

            @pl.when(c0 + 2 < NCHUNK)
            def _():
                pltpu.async_copy(
                    obs_hbm.at[:, pl.ds(chunk_base(c0 + 2), CH_C)], buf0,
                    sem0)

            bv, bi = process(c0 + 1, buf1, sem1, bv, bi)

            @pl.when(c0 + 3 < NCHUNK)
            def _():
                pltpu.async_copy(
                    obs_hbm.at[:, pl.ds(chunk_base(c0 + 3), CH_C)], buf1,
                    sem1)

            return bv, bi

        best_val, best_idx = lax.fori_loop(
            0, NCHUNK // 2, pair, (best_val, best_idx))
        if NCHUNK % 2:
            best_val, best_idx = process(
                NCHUNK - 1, buf0, sem0, best_val, best_idx)

        sval[...] = best_val
        sidx[...] = best_idx
        pltpu.sync_copy(sval, oval_hbm.at[w])
        pltpu.sync_copy(sidx, oidx_hbm.at[w])

    return stage1


def _make_tc_scan(K, TB, BN):
    TCOLS = TB * BLK
    assert TCOLS % BN == 0 and TCOLS <= K
    NSTEP = TCOLS // BN

    def scan(q_ref, blk_ref, oval_ref, oidx_ref, bestv, besti):
        i = pl.program_id(0)

        @pl.when(i == 0)
        def _():
            bestv[...] = jnp.full((1, BN), jnp.inf, jnp.float32)
            besti[...] = jnp.zeros((1, BN), jnp.int32)

        t = blk_ref[...] - q_ref[...]
        d = jnp.sum(t * t, axis=0, keepdims=True)
        idx = i * BN + lax.broadcasted_iota(jnp.int32, (1, BN), 1)
        bv = bestv[...]
        upd = d < bv
        bestv[...] = jnp.minimum(d, bv)
        besti[...] = jnp.where(upd, idx, besti[...])

        @pl.when(i == NSTEP - 1)
        def _():
            oval_ref[...] = bestv[...]
            oidx_ref[...] = besti[...]

    return pl.pallas_call(
        scan,
        grid=(NSTEP,),
        out_shape=(
            jax.ShapeDtypeStruct((1, BN), jnp.float32),
            jax.ShapeDtypeStruct((1, BN), jnp.int32),
        ),
        in_specs=[
            pl.BlockSpec((L, 1), lambda i: (0, 0)),
            pl.BlockSpec((L, BN), lambda i: (0, i)),
        ],
        out_specs=(
            pl.BlockSpec((1, BN), lambda i: (0, 0)),
            pl.BlockSpec((1, BN), lambda i: (0, 0)),
        ),
        scratch_shapes=[
            pltpu.VMEM((1, BN), jnp.float32),
            pltpu.VMEM((1, BN), jnp.int32),
        ],
        compiler_params=pltpu.CompilerParams(
            dimension_semantics=("arbitrary",)),
    )


def _make_stage2(K, NW):
    def stage2(vals_ref, idxs_ref, tval_ref, tidx_ref, pv_hbm, pm_hbm,
               ovel_ref, omask_ref, blkv, blkm, sem):
        v = vals_ref[...]
        ix = idxs_ref[...]
        m = jnp.min(v)
        ind = jnp.min(jnp.where(v == m, ix, _INT_MAX))
        tv = tval_ref[...]
        tm_ = jnp.min(tv)
        tind = jnp.min(jnp.where(tv == tm_, tidx_ref[...], _INT_MAX))
        tc_wins = (tm_ < m) | ((tm_ == m) & (tind < ind))
        ind = jnp.where(tc_wins, tind, ind)
        blk0 = (ind // BLK) * BLK
        off = ind - blk0
        pltpu.make_async_copy(
            pv_hbm.at[:, pl.ds(blk0, BLK)], blkv, sem).start()
        pltpu.make_async_copy(
            pv_hbm.at[:, pl.ds(blk0, BLK)], blkv, sem).wait()
        pltpu.make_async_copy(
            pm_hbm.at[:, pl.ds(blk0, BLK)], blkm, sem).start()
        pltpu.make_async_copy(
            pm_hbm.at[:, pl.ds(blk0, BLK)], blkm, sem).wait()
        lane = lax.broadcasted_iota(jnp.int32, (L, BLK), 1)
        sel = lane == off
        rowv = jnp.sum(jnp.where(sel, blkv[...], 0.0), axis=1)
        rowm = jnp.sum(jnp.where(sel, blkm[...], 0.0), axis=1)
        ovel_ref[...] = rowv.reshape(1, L)
        omask_ref[...] = rowm.reshape(1, L)

    return pl.pallas_call(
        stage2,
        out_shape=(
            jax.ShapeDtypeStruct((1, L), jnp.float32),
            jax.ShapeDtypeStruct((1, L), jnp.float32),
        ),
        in_specs=[
            pl.BlockSpec(memory_space=pltpu.MemorySpace.VMEM),
            pl.BlockSpec(memory_space=pltpu.MemorySpace.VMEM),
            pl.BlockSpec(memory_space=pltpu.MemorySpace.VMEM),
            pl.BlockSpec(memory_space=pltpu.MemorySpace.VMEM),
            pl.BlockSpec(memory_space=pl.ANY),
            pl.BlockSpec(memory_space=pl.ANY),
        ],
        out_specs=(
            pl.BlockSpec(memory_space=pltpu.MemorySpace.VMEM),
            pl.BlockSpec(memory_space=pltpu.MemorySpace.VMEM),
        ),
        scratch_shapes=[
            pltpu.VMEM((L, BLK), jnp.float32),
            pltpu.VMEM((L, BLK), jnp.float32),
            pltpu.SemaphoreType.DMA,
        ],
        compiler_params=pltpu.CompilerParams(disable_bounds_checks=True),
    )


def kernel(in_vel, obs_vel, pred_vel, pred_mask):
    K, D = obs_vel.shape
    assert D == L
    info = plsc.get_sparse_core_info()
    NW = info.num_cores * info.num_subcores
    CH_B = 16
    TB = 1440

    obs_t = obs_vel.T
    pv_t = pred_vel.T
    pm_t = pred_mask.T

    vals, idxs = _make_stage1(K, NW, CH_B, TB)(in_vel, obs_t)
    tval, tidx = _make_tc_scan(K, TB, 4096)(in_vel.T, obs_t)
    best_vel, best_mask = _make_stage2(K, NW)(
        vals, idxs, tval, tidx, pv_t, pm_t)
    return best_vel, best_mask


# speedup vs baseline: 4.3413x; 1.0084x over previous
import functools

import jax
import jax.numpy as jnp
from jax import lax
from jax.experimental import pallas as pl
from jax.experimental.pallas import tpu as pltpu
from jax.experimental.pallas import tpu_sc as plsc

L = 16
BLK = 128

_INT_MAX = 2**31 - 1

_SC_PARAMS = dict(
    needs_layout_passes=False,
    use_tc_tiling_on_sc=True,
    disable_bounds_checks=True,
)


def _worker_id():
    return lax.axis_index("s") * lax.axis_size("c") + lax.axis_index("c")


def _take16(v, idx):
    return lax.gather(
        v, idx[:, None],
        dimension_numbers=lax.GatherDimensionNumbers(
            offset_dims=(), collapsed_slice_dims=(0,), start_index_map=(0,)),
        slice_sizes=(1,),
        mode=lax.GatherScatterMode.PROMISE_IN_BOUNDS)


def _better(val, idx, best_val, best_idx):
    upd = (val < best_val) | ((val == best_val) & (idx < best_idx))
    return jnp.where(upd, val, best_val), jnp.where(upd, idx, best_idx)


def _make_stage1(K, NW, CH_B, TB):
    NB = -(-K // BLK)
    NBW = NB - TB
    BASE_B = NBW // NW
    EXTRA = NBW % NW
    NCHUNK = -(-(BASE_B + (1 if EXTRA else 0)) // CH_B)
    CH_C = CH_B * BLK

    mesh = plsc.VectorSubcoreMesh(core_axis_name="c", subcore_axis_name="s")

    @functools.partial(
        pl.kernel,
        out_type=(
            jax.ShapeDtypeStruct((NW, L), jnp.float32),
            jax.ShapeDtypeStruct((NW, L), jnp.int32),
        ),
        mesh=mesh,
        compiler_params=pltpu.CompilerParams(**_SC_PARAMS),
        scratch_types=[
            pltpu.VMEM((L, CH_C), jnp.float32),
            pltpu.VMEM((L, CH_C), jnp.float32),
            pltpu.VMEM((1, L), jnp.float32),
            pltpu.VMEM((L,), jnp.float32),
            pltpu.VMEM((L,), jnp.int32),
            pltpu.SemaphoreType.DMA,
            pltpu.SemaphoreType.DMA,
        ],
    )
    def stage1(q_hbm, obs_hbm, oval_hbm, oidx_hbm,
               buf0, buf1, qv, sval, sidx, sem0, sem1):
        w = _worker_id()
        b0 = TB + w * BASE_B + jnp.minimum(w, EXTRA)
        nb = BASE_B + jnp.where(w < EXTRA, 1, 0)
        col0 = b0 * BLK
        col_hi = (b0 + nb) * BLK - CH_C

        pltpu.sync_copy(q_hbm, qv)
        qvec = qv[0]
        qs = [jnp.full((L,), qvec[f], jnp.float32) for f in range(L)]

        iota = lax.iota(jnp.int32, L)

        bufs = (buf0, buf1)
        sems = (sem0, sem1)

        def chunk_base(c):
            return jnp.minimum(col0 + c * CH_C, col_hi)

        def start(c):
            return pltpu.async_copy(
                obs_hbm.at[:, pl.ds(chunk_base(c), CH_C)], bufs[c % 2],
                sems[c % 2])

        best_val = jnp.full((L,), jnp.inf, jnp.float32)
        best_idx = jnp.zeros((L,), jnp.int32)

        inf16 = jnp.full((L,), jnp.inf, jnp.float32)

        def process(c, buf, sem, bv, bi):
            pltpu.make_async_copy(
                obs_hbm.at[:, pl.ds(chunk_base(c), CH_C)], buf, sem).wait()
            cb = chunk_base(c)

            @pl.when(cb + CH_C > K)
            def _():
                def poison(t, _):
                    for f in range(L):
                        buf[f, pl.ds(K - cb + t * L, L)] = inf16
                    return 0
                lax.fori_loop(0, (cb + CH_C - K) // L, poison, 0)

            base_idx = cb + iota

            def group(g, carry):
                gv, gi = carry
                r = g * L
                parts = []
                for f in range(L):
                    t = buf[f, pl.ds(r, L)] - qs[f]
                    parts.append(t * t)
                while len(parts) > 1:
                    parts = [parts[i] + parts[i + 1]
                             for i in range(0, len(parts), 2)]
                dist = parts[0]
                upd = dist < gv
                return (jnp.minimum(dist, gv),
                        jnp.where(upd, base_idx + r, gi))

            return lax.fori_loop(0, CH_C // L, group, (bv, bi))

        cp0 = start(0)
        cp1 = start(1)

        def pair(i, carry):
            bv, bi = carry
            c0 = 2 * i
            bv, bi = process(c0, buf0, sem0, bv, bi)

            @pl.when(c0 + 2 < NCHUNK)
            def _():
                pltpu.async_copy(
                    obs_hbm.at[:, pl.ds(chunk_base(c0 + 2), CH_C)], buf0,
                    sem0)

            bv, bi = process(c0 + 1, buf1, sem1, bv, bi)

            @pl.when(c0 + 3 < NCHUNK)
            def _():
                pltpu.async_copy(
                    obs_hbm.at[:, pl.ds(chunk_base(c0 + 3), CH_C)], buf1,
                    sem1)

            return bv, bi

        best_val, best_idx = lax.fori_loop(
            0, NCHUNK // 2, pair, (best_val, best_idx))
        if NCHUNK % 2:
            best_val, best_idx = process(
                NCHUNK - 1, buf0, sem0, best_val, best_idx)

        sval[...] = best_val
        sidx[...] = best_idx
        pltpu.sync_copy(sval, oval_hbm.at[w])
        pltpu.sync_copy(sidx, oidx_hbm.at[w])

    return stage1


def _make_tc_scan(K, TB, BN):
    TCOLS = TB * BLK
    assert TCOLS % BN == 0 and TCOLS <= K
    NSTEP = TCOLS // BN

    def scan(q_ref, blk_ref, oval_ref, oidx_ref, bestv, besti):
        i = pl.program_id(0)

        @pl.when(i == 0)
        def _():
            bestv[...] = jnp.full((1, BN), jnp.inf, jnp.float32)
            besti[...] = jnp.zeros((1, BN), jnp.int32)

        t = blk_ref[...] - q_ref[...]
        d = jnp.sum(t * t, axis=0, keepdims=True)
        idx = i * BN + lax.broadcasted_iota(jnp.int32, (1, BN), 1)
        bv = bestv[...]
        upd = d < bv
        bestv[...] = jnp.minimum(d, bv)
        besti[...] = jnp.where(upd, idx, besti[...])

        @pl.when(i == NSTEP - 1)
        def _():
            oval_ref[...] = bestv[...]
            oidx_ref[...] = besti[...]

    return pl.pallas_call(
        scan,
        grid=(NSTEP,),
        out_shape=(
            jax.ShapeDtypeStruct((1, BN), jnp.float32),
            jax.ShapeDtypeStruct((1, BN), jnp.int32),
        ),
        in_specs=[
            pl.BlockSpec((L, 1), lambda i: (0, 0)),
            pl.BlockSpec((L, BN), lambda i: (0, i)),
        ],
        out_specs=(
            pl.BlockSpec((1, BN), lambda i: (0, 0)),
            pl.BlockSpec((1, BN), lambda i: (0, 0)),
        ),
        scratch_shapes=[
            pltpu.VMEM((1, BN), jnp.float32),
            pltpu.VMEM((1, BN), jnp.int32),
        ],
        compiler_params=pltpu.CompilerParams(
            dimension_semantics=("arbitrary",)),
    )


def _make_stage2(K, NW):
    def stage2(vals_ref, idxs_ref, tval_ref, tidx_ref, pv_hbm, pm_hbm,
               ovel_ref, omask_ref, blkv, blkm, sem):
        v = vals_ref[...]
        ix = idxs_ref[...]
        m = jnp.min(v)
        ind = jnp.min(jnp.where(v == m, ix, _INT_MAX))
        tv = tval_ref[...]
        tm_ = jnp.min(tv)
        tind = jnp.min(jnp.where(tv == tm_, tidx_ref[...], _INT_MAX))
        tc_wins = (tm_ < m) | ((tm_ == m) & (tind < ind))
        ind = jnp.where(tc_wins, tind, ind)
        blk0 = (ind // BLK) * BLK
        off = ind - blk0
        pltpu.make_async_copy(
            pv_hbm.at[:, pl.ds(blk0, BLK)], blkv, sem).start()
        pltpu.make_async_copy(
            pv_hbm.at[:, pl.ds(blk0, BLK)], blkv, sem).wait()
        pltpu.make_async_copy(
            pm_hbm.at[:, pl.ds(blk0, BLK)], blkm, sem).start()
        pltpu.make_async_copy(
            pm_hbm.at[:, pl.ds(blk0, BLK)], blkm, sem).wait()
        lane = lax.broadcasted_iota(jnp.int32, (L, BLK), 1)
        sel = lane == off
        rowv = jnp.sum(jnp.where(sel, blkv[...], 0.0), axis=1)
        rowm = jnp.sum(jnp.where(sel, blkm[...], 0.0), axis=1)
        ovel_ref[...] = rowv.reshape(1, L)
        omask_ref[...] = rowm.reshape(1, L)

    return pl.pallas_call(
        stage2,
        out_shape=(
            jax.ShapeDtypeStruct((1, L), jnp.float32),
            jax.ShapeDtypeStruct((1, L), jnp.float32),
        ),
        in_specs=[
            pl.BlockSpec(memory_space=pltpu.MemorySpace.VMEM),
            pl.BlockSpec(memory_space=pltpu.MemorySpace.VMEM),
            pl.BlockSpec(memory_space=pltpu.MemorySpace.VMEM),
            pl.BlockSpec(memory_space=pltpu.MemorySpace.VMEM),
            pl.BlockSpec(memory_space=pl.ANY),
            pl.BlockSpec(memory_space=pl.ANY),
        ],
        out_specs=(
            pl.BlockSpec(memory_space=pltpu.MemorySpace.VMEM),
            pl.BlockSpec(memory_space=pltpu.MemorySpace.VMEM),
        ),
        scratch_shapes=[
            pltpu.VMEM((L, BLK), jnp.float32),
            pltpu.VMEM((L, BLK), jnp.float32),
            pltpu.SemaphoreType.DMA,
        ],
        compiler_params=pltpu.CompilerParams(disable_bounds_checks=True),
    )


def kernel(in_vel, obs_vel, pred_vel, pred_mask):
    K, D = obs_vel.shape
    assert D == L
    info = plsc.get_sparse_core_info()
    NW = info.num_cores * info.num_subcores
    CH_B = 16
    TB = 1312

    obs_t = obs_vel.T
    pv_t = pred_vel.T
    pm_t = pred_mask.T

    vals, idxs = _make_stage1(K, NW, CH_B, TB)(in_vel, obs_t)
    tval, tidx = _make_tc_scan(K, TB, 4096)(in_vel.T, obs_t)
    best_vel, best_mask = _make_stage2(K, NW)(
        vals, idxs, tval, tidx, pv_t, pm_t)
    return best_vel, best_mask
